# baseline (speedup 1.0000x reference)
import functools

import jax
import jax.numpy as jnp
from jax import lax
from jax.experimental import pallas as pl
from jax.experimental.pallas import tpu as pltpu
from jax.experimental.pallas import tpu_sc as plsc

_LANES = 16


def _make_sc_kernel(B, L, D, V):
    info = plsc.get_sparse_core_info()
    NC, NS = info.num_cores, info.num_subcores
    NW = NC * NS
    assert B % NW == 0, (B, NW)
    seqs_per_w = B // NW
    assert L % 2 == 0 and D % _LANES == 0
    H = L // 2

    mesh = plsc.VectorSubcoreMesh(core_axis_name="c", subcore_axis_name="s")

    @functools.partial(
        pl.kernel,
        out_type=jax.ShapeDtypeStruct((B, L, D), jnp.float32),
        mesh=mesh,
        compiler_params=pltpu.CompilerParams(use_tc_tiling_on_sc=False),
        scratch_types=[
            pltpu.VMEM((2, H), jnp.int32),
            pltpu.VMEM((L + _LANES,), jnp.float32),
            pltpu.VMEM((L, D), jnp.float32),
            pltpu.VMEM((L, D), jnp.float32),
            pltpu.VMEM((D,), jnp.float32),
            pltpu.SemaphoreType.DMA,
        ],
    )
    def emb_kernel(idx_hbm, mask_hbm, table_hbm, me_hbm, pos_hbm, out_hbm,
                   idx_v, mask_v, rows_v, pos_v, me_v, sem):
        wid = lax.axis_index("s") * NC + lax.axis_index("c")

        pltpu.sync_copy(pos_hbm, pos_v)
        pltpu.sync_copy(me_hbm, me_v)
        me_chunks = [me_v[pl.ds(k * _LANES, _LANES)] for k in range(D // _LANES)]

        def row_body(r, carry):
            m = mask_v[pl.ds(r, _LANES)][0]
            mv = jnp.full((_LANES,), m, dtype=jnp.float32)
            omv = 1.0 - mv
            for k in range(D // _LANES):
                sl = pl.ds(k * _LANES, _LANES)
                g = rows_v[r, sl]
                p = pos_v[r, sl]
                rows_v[r, sl] = (mv * g + omv * me_chunks[k]) + p
            return carry

        def seq_body(s, carry):
            b = wid * seqs_per_w + s
            pltpu.sync_copy(idx_hbm.at[b], idx_v)
            pltpu.sync_copy(mask_hbm.at[b], mask_v)
            cp0 = pltpu.async_copy(table_hbm.at[idx_v.at[0]],
                                   rows_v.at[pl.ds(0, H)], sem)
            cp1 = pltpu.async_copy(table_hbm.at[idx_v.at[1]],
                                   rows_v.at[pl.ds(H, H)], sem)
            cp0.wait()
            cp1.wait()
            lax.fori_loop(0, L, row_body, None)
            pltpu.sync_copy(rows_v, out_hbm.at[b])
            return carry

        lax.fori_loop(0, seqs_per_w, seq_body, None)

    return emb_kernel


def kernel(item_id, token_mask, item_table, mask_emb, pos_emb):
    B, L = item_id.shape
    V, D = item_table.shape
    idx3 = item_id.astype(jnp.int32).reshape(B, 2, L // 2)
    mask_f = jnp.pad(token_mask.astype(jnp.float32), ((0, 0), (0, _LANES)))
    me = mask_emb.reshape(D)
    emb = _make_sc_kernel(B, L, D, V)
    return emb(idx3, mask_f, item_table, me, pos_emb)


# speedup vs baseline: 1.3412x; 1.3412x over previous
import functools

import jax
import jax.numpy as jnp
from jax import lax
from jax.experimental import pallas as pl
from jax.experimental.pallas import tpu as pltpu
from jax.experimental.pallas import tpu_sc as plsc

_LANES = 16
_RING = 4

_IDX0_OFF = 0
_IDX1_OFF = 104
_MASK_OFF = 208
_GLEN = 104
_H1_BASE = 96
_LINE = 416


def _make_sc_kernel(B, L, D):
    info = plsc.get_sparse_core_info()
    NC, NS = info.num_cores, info.num_subcores
    NW = NC * NS
    assert B % NW == 0, (B, NW)
    SPW = B // NW
    assert L == 200 and D % _LANES == 0
    NBLK = L // _LANES
    TAIL = L - NBLK * _LANES
    DCH = D // _LANES

    mesh = plsc.VectorSubcoreMesh(core_axis_name="c", subcore_axis_name="s")

    @functools.partial(
        pl.kernel,
        out_type=jax.ShapeDtypeStruct((B, L, D), jnp.float32),
        mesh=mesh,
        compiler_params=pltpu.CompilerParams(use_tc_tiling_on_sc=False),
        scratch_types=[
            pltpu.VMEM((_RING, _LINE), jnp.int32),
            pltpu.VMEM((_RING, L, D), jnp.float32),
            pltpu.VMEM((L, D), jnp.float32),
            pltpu.VMEM((D,), jnp.float32),
            pltpu.SemaphoreType.DMA((_RING,)),
            pltpu.SemaphoreType.DMA((_RING,)),
            pltpu.SemaphoreType.DMA((_RING,)),
        ],
    )
    def emb_kernel(line_hbm, table_hbm, me_hbm, pos_hbm, out_hbm,
                   line_v, rows_v, pos_v, me_v, psem, gsem, osem):
        wid = lax.axis_index("s") * NC + lax.axis_index("c")
        base = wid * SPW

        pltpu.sync_copy(pos_hbm, pos_v)
        pltpu.sync_copy(me_hbm, me_v)
        me_chunks = [me_v[pl.ds(k * _LANES, _LANES)] for k in range(DCH)]

        def start_line(s, slot):
            return pltpu.async_copy(line_hbm.at[base + s], line_v.at[slot],
                                    psem.at[slot])

        def gather_copies(s, slot, issue):
            fn = pltpu.async_copy if issue else pltpu.make_async_copy
            c0 = fn(table_hbm.at[line_v.at[slot, pl.ds(_IDX0_OFF, _GLEN)]],
                    rows_v.at[slot].at[pl.ds(0, _GLEN)], gsem.at[slot])
            c1 = fn(table_hbm.at[line_v.at[slot, pl.ds(_IDX1_OFF, _GLEN)]],
                    rows_v.at[slot].at[pl.ds(_H1_BASE, _GLEN)], gsem.at[slot])
            return c0, c1

        def do_rows(slot, t, nrows):
            mchunk = line_v[slot, pl.ds(_MASK_OFF + t * _LANES, _LANES)]
            for j in range(nrows):
                r = t * _LANES + j
                mv = jnp.full((_LANES,), mchunk[j],
                              dtype=jnp.int32).astype(jnp.float32)
                omv = 1.0 - mv
                for k in range(DCH):
                    sl = pl.ds(k * _LANES, _LANES)
                    g = rows_v[slot, r, sl]
                    p = pos_v[r, sl]
                    rows_v[slot, r, sl] = (mv * g + omv * me_chunks[k]) + p

        def seq_step(s, q):
            w0, w1 = gather_copies(s, q, issue=False)
            w0.wait()
            w1.wait()

            def blk(t, carry):
                do_rows(q, t, _LANES)
                return carry
            lax.fori_loop(0, NBLK, blk, None)
            do_rows(q, NBLK, TAIL)

            pltpu.async_copy(rows_v.at[q], out_hbm.at[base + s], osem.at[q])

            @pl.when(s + 3 < SPW)
            def _():
                start_line(s + 3, (q + 3) % _RING)

            @pl.when(s + 2 < SPW)
            def _():
                q2 = (q + 2) % _RING

                @pl.when(s >= 1)
                def _():
                    pltpu.make_async_copy(line_hbm.at[base + s + 2],
                                          line_v.at[q2], psem.at[q2]).wait()

                @pl.when(s >= 2)
                def _():
                    ow = pltpu.make_async_copy(rows_v.at[q2],
                                               out_hbm.at[base + s - 2],
                                               osem.at[q2])
                    ow.wait()
                gather_copies(s + 2, q2, issue=True)

        for s0 in range(3):
            pltpu.sync_copy(line_hbm.at[base + s0], line_v.at[s0])
        gather_copies(0, 0, issue=True)
        gather_copies(1, 1, issue=True)

        def body(kk, carry):
            for q in range(_RING):
                seq_step(kk * _RING + q, q)
            return carry
        lax.fori_loop(0, SPW // _RING, body, None)

        for s0 in (SPW - 2, SPW - 1):
            q = s0 % _RING
            pltpu.make_async_copy(rows_v.at[q], out_hbm.at[base + s0],
                                  osem.at[q]).wait()

    return emb_kernel


def kernel(item_id, token_mask, item_table, mask_emb, pos_emb):
    B, L = item_id.shape
    V, D = item_table.shape
    idx = item_id.astype(jnp.int32)
    maskbits = token_mask.astype(jnp.int32)
    line = jnp.concatenate(
        [idx[:, :_GLEN], idx[:, _H1_BASE:], maskbits,
         jnp.zeros((B, _LINE - _MASK_OFF - L), jnp.int32)], axis=1)
    me = mask_emb.reshape(D)
    emb = _make_sc_kernel(B, L, D)
    return emb(line, item_table, me, pos_emb)
